# baseline (speedup 1.0000x reference)
import functools

import jax
import jax.numpy as jnp
from jax import lax
from jax.experimental import pallas as pl
from jax.experimental.pallas import tpu as pltpu
from jax.experimental.pallas import tpu_sc as plsc


def _tr_body(src, dst):
    eye = jnp.eye(src.shape[0], dtype=jnp.float32)
    dst[...] = jax.lax.dot_general(
        src[...], eye, (((0,), (0,)), ((), ())),
        preferred_element_type=jnp.float32)


@functools.lru_cache(maxsize=None)
def _make_transpose(D, V, BLK):
    grid = (V + BLK - 1) // BLK
    return pl.pallas_call(
        _tr_body,
        grid=(grid,),
        in_specs=[pl.BlockSpec((D, BLK), lambda i: (0, i))],
        out_specs=pl.BlockSpec((BLK, D), lambda i: (i, 0)),
        out_shape=jax.ShapeDtypeStruct((grid * BLK, D), jnp.float32),
    )


_CHUNK = 16


@functools.lru_cache(maxsize=None)
def _make_gather(B, D):
    info = plsc.get_sparse_core_info()
    NC, NS = info.num_cores, info.num_subcores
    NW = NC * NS
    assert B % NW == 0
    b_per_w = B // NW
    assert b_per_w % _CHUNK == 0

    mesh = plsc.VectorSubcoreMesh(core_axis_name="c", subcore_axis_name="s")

    @functools.partial(
        pl.kernel,
        mesh=mesh,
        out_type=(
            jax.ShapeDtypeStruct((B, D), jnp.float32),
            jax.ShapeDtypeStruct((B, D), jnp.float32),
        ),
        scratch_types=[
            pltpu.VMEM((b_per_w,), jnp.int32),
            pltpu.VMEM((b_per_w, D), jnp.float32),
            pltpu.SemaphoreType.DMA,
        ],
    )
    def gather(uid_hbm, pid_hbm, utab_hbm, ptab_hbm, uout_hbm, pout_hbm,
               idx_s, rows_v, sem):
        wid = lax.axis_index("s") * NC + lax.axis_index("c")
        base = wid * b_per_w

        def one_table(id_hbm, tab_hbm, out_hbm):
            pltpu.sync_copy(id_hbm.at[pl.ds(base, b_per_w)], idx_s)

            def chunk(c, _):
                off = c * _CHUNK
                idx16 = idx_s[pl.ds(off, _CHUNK)]
                cps = [
                    pltpu.async_copy(
                        tab_hbm.at[pl.ds(idx16[j], 1)],
                        rows_v.at[pl.ds(off + j, 1)], sem)
                    for j in range(_CHUNK)
                ]
                for cp in cps:
                    cp.wait()
                return ()

            lax.fori_loop(0, b_per_w // _CHUNK, chunk, ())
            pltpu.sync_copy(rows_v, out_hbm.at[pl.ds(base, b_per_w)])

        one_table(uid_hbm, utab_hbm, uout_hbm)
        one_table(pid_hbm, ptab_hbm, pout_hbm)

    return gather


def _mlp_body(u, p, w1u, w1p, b1, w2, b2, w3, b3, out):
    h1 = jnp.dot(u[...], w1u[...], preferred_element_type=jnp.float32)
    h1 += jnp.dot(p[...], w1p[...], preferred_element_type=jnp.float32)
    h1 = jnp.maximum(h1 + b1[...], 0.0)
    h2 = jnp.maximum(
        jnp.dot(h1, w2[...], preferred_element_type=jnp.float32) + b2[...], 0.0)
    out[...] = jnp.dot(h2, w3[...], preferred_element_type=jnp.float32) + b3[...]


@functools.lru_cache(maxsize=None)
def _make_mlp(B, D, H1, H2, BLK):
    grid = B // BLK
    full = lambda i: (0, 0)
    return pl.pallas_call(
        _mlp_body,
        grid=(grid,),
        in_specs=[
            pl.BlockSpec((BLK, D), lambda i: (i, 0)),
            pl.BlockSpec((BLK, D), lambda i: (i, 0)),
            pl.BlockSpec((D, H1), full),
            pl.BlockSpec((D, H1), full),
            pl.BlockSpec((1, H1), full),
            pl.BlockSpec((H1, H2), full),
            pl.BlockSpec((1, H2), full),
            pl.BlockSpec((H2, 1), full),
            pl.BlockSpec((1, 1), full),
        ],
        out_specs=pl.BlockSpec((BLK, 1), lambda i: (i, 0)),
        out_shape=jax.ShapeDtypeStruct((B, 1), jnp.float32),
    )


def kernel(userId, productId, user_table, product_table, W1, b1, W2, b2, W3, b3):
    B = userId.shape[0]
    VU, D = user_table.shape
    VP = product_table.shape[0]
    H1 = W1.shape[1]
    H2 = W2.shape[1]

    utab = _make_transpose(D, VU, 2048)(user_table.T)
    ptab = _make_transpose(D, VP, 2048)(product_table.T)

    gather = _make_gather(B, D)
    u_emb, p_emb = gather(userId.astype(jnp.int32), productId.astype(jnp.int32),
                          utab, ptab)

    mlp = _make_mlp(B, D, H1, H2, BLK=2048)
    return mlp(u_emb, p_emb, W1[:D], W1[D:], b1[None, :], W2, b2[None, :],
               W3, b3[None, :])


# speedup vs baseline: 1.4603x; 1.4603x over previous
import functools

import jax
import jax.numpy as jnp
from jax import lax
from jax.experimental import pallas as pl
from jax.experimental.pallas import tpu as pltpu
from jax.experimental.pallas import tpu_sc as plsc

_PACK = 4
_PBLK = 2048
_CHUNK = 16


def _pack_body(s0, s1, s2, s3, dst):
    for a, s in enumerate((s0, s1, s2, s3)):
        eye = jnp.eye(s.shape[0], dtype=jnp.float32)
        dst[:, a * s.shape[0]:(a + 1) * s.shape[0]] = jax.lax.dot_general(
            s[...], eye, (((0,), (0,)), ((), ())),
            preferred_element_type=jnp.float32)


@functools.lru_cache(maxsize=None)
def _make_pack(D, V):
    grid = (V + _PACK * _PBLK - 1) // (_PACK * _PBLK)
    Q = grid * _PBLK
    last = (V + _PBLK - 1) // _PBLK - 1

    def _imap(a, i):
        return (0, jnp.minimum(i + a * grid, last))

    specs = [
        pl.BlockSpec((D, _PBLK), functools.partial(_imap, a))
        for a in range(_PACK)
    ]
    call = pl.pallas_call(
        _pack_body,
        grid=(grid,),
        in_specs=specs,
        out_specs=pl.BlockSpec((_PBLK, _PACK * D), lambda i: (i, 0)),
        out_shape=jax.ShapeDtypeStruct((Q, _PACK * D), jnp.float32),
    )

    def pack(tabT):
        return call(tabT, tabT, tabT, tabT)

    return pack, Q


@functools.lru_cache(maxsize=None)
def _make_gather(B, W):
    info = plsc.get_sparse_core_info()
    NC, NS = info.num_cores, info.num_subcores
    NW = NC * NS
    assert B % NW == 0
    b_per_w = B // NW
    assert b_per_w % _CHUNK == 0

    mesh = plsc.VectorSubcoreMesh(core_axis_name="c", subcore_axis_name="s")

    @functools.partial(
        pl.kernel,
        mesh=mesh,
        out_type=(
            jax.ShapeDtypeStruct((B, W), jnp.float32),
            jax.ShapeDtypeStruct((B, W), jnp.float32),
        ),
        scratch_types=[
            pltpu.VMEM((b_per_w,), jnp.int32),
            pltpu.VMEM((b_per_w, W), jnp.float32),
            pltpu.SemaphoreType.DMA,
        ],
    )
    def gather(uid_hbm, pid_hbm, utab_hbm, ptab_hbm, uout_hbm, pout_hbm,
               idx_s, rows_v, sem):
        wid = lax.axis_index("s") * NC + lax.axis_index("c")
        base = wid * b_per_w

        def one_table(id_hbm, tab_hbm, out_hbm):
            pltpu.sync_copy(id_hbm.at[pl.ds(base, b_per_w)], idx_s)

            def chunk(c, _):
                off = c * _CHUNK
                idx16 = idx_s[pl.ds(off, _CHUNK)]
                cps = [
                    pltpu.async_copy(
                        tab_hbm.at[pl.ds(idx16[j], 1)],
                        rows_v.at[pl.ds(off + j, 1)], sem)
                    for j in range(_CHUNK)
                ]
                for cp in cps:
                    cp.wait()
                return ()

            lax.fori_loop(0, b_per_w // _CHUNK, chunk, ())
            pltpu.sync_copy(rows_v, out_hbm.at[pl.ds(base, b_per_w)])

        one_table(uid_hbm, utab_hbm, uout_hbm)
        one_table(pid_hbm, ptab_hbm, pout_hbm)

    return gather


def _select(w, a, D):
    x = jnp.where(a == 0, w[:, 0:D], 0.0)
    for k in range(1, _PACK):
        x += jnp.where(a == k, w[:, k * D:(k + 1) * D], 0.0)
    return x


def _mlp_body(uw, pw, ua, pa, w1u, w1p, b1, w2, b2, w3, b3, out):
    D = w1u.shape[0]
    u = _select(uw[...], ua[...], D)
    p = _select(pw[...], pa[...], D)
    h1 = jnp.dot(u, w1u[...], preferred_element_type=jnp.float32)
    h1 += jnp.dot(p, w1p[...], preferred_element_type=jnp.float32)
    h1 = jnp.maximum(h1 + b1[...], 0.0)
    h2 = jnp.maximum(
        jnp.dot(h1, w2[...], preferred_element_type=jnp.float32) + b2[...], 0.0)
    out[...] = jnp.dot(h2, w3[...], preferred_element_type=jnp.float32) + b3[...]


@functools.lru_cache(maxsize=None)
def _make_mlp(B, W, D, H1, H2, BLK):
    grid = B // BLK
    full = lambda i: (0, 0)
    return pl.pallas_call(
        _mlp_body,
        grid=(grid,),
        in_specs=[
            pl.BlockSpec((BLK, W), lambda i: (i, 0)),
            pl.BlockSpec((BLK, W), lambda i: (i, 0)),
            pl.BlockSpec((BLK, 1), lambda i: (i, 0)),
            pl.BlockSpec((BLK, 1), lambda i: (i, 0)),
            pl.BlockSpec((D, H1), full),
            pl.BlockSpec((D, H1), full),
            pl.BlockSpec((1, H1), full),
            pl.BlockSpec((H1, H2), full),
            pl.BlockSpec((1, H2), full),
            pl.BlockSpec((H2, 1), full),
            pl.BlockSpec((1, 1), full),
        ],
        out_specs=pl.BlockSpec((BLK, 1), lambda i: (i, 0)),
        out_shape=jax.ShapeDtypeStruct((B, 1), jnp.float32),
    )


def kernel(userId, productId, user_table, product_table, W1, b1, W2, b2, W3, b3):
    B = userId.shape[0]
    VU, D = user_table.shape
    VP = product_table.shape[0]
    H1 = W1.shape[1]
    H2 = W2.shape[1]
    W = _PACK * D

    upack, QU = _make_pack(D, VU)
    ppack, QP = _make_pack(D, VP)
    utab = upack(user_table.T)
    ptab = ppack(product_table.T)

    uid = userId.astype(jnp.int32)
    pid = productId.astype(jnp.int32)
    uw, pw = _make_gather(B, W)(uid % QU, pid % QP, utab, ptab)

    mlp = _make_mlp(B, W, D, H1, H2, BLK=2048)
    return mlp(uw, pw, (uid // QU)[:, None], (pid // QP)[:, None],
               W1[:D], W1[D:], b1[None, :], W2, b2[None, :], W3, b3[None, :])


# speedup vs baseline: 1.8278x; 1.2516x over previous
import functools

import jax
import jax.numpy as jnp
from jax import lax
from jax.experimental import pallas as pl
from jax.experimental.pallas import tpu as pltpu
from jax.experimental.pallas import tpu_sc as plsc

_PACK = 4
_PBLK = 2048
_CHUNK = 16


def _pack_body(s0, s1, s2, s3, dst):
    for a, s in enumerate((s0, s1, s2, s3)):
        eye = jnp.eye(s.shape[0], dtype=jnp.bfloat16)
        dst[:, a * s.shape[0]:(a + 1) * s.shape[0]] = jax.lax.dot_general(
            s[...].astype(jnp.bfloat16), eye, (((0,), (0,)), ((), ())),
            preferred_element_type=jnp.float32)


@functools.lru_cache(maxsize=None)
def _make_pack(D, V):
    grid = (V + _PACK * _PBLK - 1) // (_PACK * _PBLK)
    Q = grid * _PBLK
    last = (V + _PBLK - 1) // _PBLK - 1

    def _imap(a, i):
        return (0, jnp.minimum(i + a * grid, last))

    specs = [
        pl.BlockSpec((D, _PBLK), functools.partial(_imap, a))
        for a in range(_PACK)
    ]
    call = pl.pallas_call(
        _pack_body,
        grid=(grid,),
        in_specs=specs,
        out_specs=pl.BlockSpec((_PBLK, _PACK * D), lambda i: (i, 0)),
        out_shape=jax.ShapeDtypeStruct((Q, _PACK * D), jnp.float32),
    )

    def pack(tabT):
        return call(tabT, tabT, tabT, tabT)

    return pack, Q


@functools.lru_cache(maxsize=None)
def _make_gather(B, W):
    info = plsc.get_sparse_core_info()
    NC, NS = info.num_cores, info.num_subcores
    NW = NC * NS
    assert B % NW == 0
    b_per_w = B // NW
    assert b_per_w % _CHUNK == 0

    mesh = plsc.VectorSubcoreMesh(core_axis_name="c", subcore_axis_name="s")

    @functools.partial(
        pl.kernel,
        mesh=mesh,
        out_type=(
            jax.ShapeDtypeStruct((B, W), jnp.float32),
            jax.ShapeDtypeStruct((B, W), jnp.float32),
        ),
        scratch_types=[
            pltpu.VMEM((b_per_w,), jnp.int32),
            pltpu.VMEM((b_per_w, W), jnp.float32),
            pltpu.SemaphoreType.DMA,
        ],
    )
    def gather(uid_hbm, pid_hbm, utab_hbm, ptab_hbm, uout_hbm, pout_hbm,
               idx_s, rows_v, sem):
        wid = lax.axis_index("s") * NC + lax.axis_index("c")
        base = wid * b_per_w

        nchunk = b_per_w // _CHUNK

        def one_table(id_hbm, tab_hbm, out_hbm):
            pltpu.sync_copy(id_hbm.at[pl.ds(base, b_per_w)], idx_s)

            def fire(off):
                idx16 = idx_s[pl.ds(off, _CHUNK)]
                for j in range(_CHUNK):
                    pltpu.async_copy(
                        tab_hbm.at[pl.ds(idx16[j], 1)],
                        rows_v.at[pl.ds(off + j, 1)], sem)

            def drain(off):
                pltpu.make_async_copy(
                    tab_hbm.at[pl.ds(0, _CHUNK)],
                    rows_v.at[pl.ds(off, _CHUNK)], sem).wait()

            fire(0)

            def chunk(c, _):
                fire(c * _CHUNK)
                drain((c - 1) * _CHUNK)
                return ()

            lax.fori_loop(1, nchunk, chunk, ())
            drain((nchunk - 1) * _CHUNK)
            pltpu.sync_copy(rows_v, out_hbm.at[pl.ds(base, b_per_w)])

        one_table(uid_hbm, utab_hbm, uout_hbm)
        one_table(pid_hbm, ptab_hbm, pout_hbm)

    return gather


def _select(w, a, D):
    x = jnp.where(a == 0, w[:, 0:D], 0.0)
    for k in range(1, _PACK):
        x += jnp.where(a == k, w[:, k * D:(k + 1) * D], 0.0)
    return x


def _mlp_body(uw, pw, ua, pa, w1u, w1p, b1, w2, b2, w3, b3, out):
    D = w1u.shape[0]
    u = _select(uw[...], ua[...], D)
    p = _select(pw[...], pa[...], D)
    h1 = jnp.dot(u, w1u[...], preferred_element_type=jnp.float32)
    h1 += jnp.dot(p, w1p[...], preferred_element_type=jnp.float32)
    h1 = jnp.maximum(h1 + b1[...], 0.0)
    h2 = jnp.maximum(
        jnp.dot(h1, w2[...], preferred_element_type=jnp.float32) + b2[...], 0.0)
    out[...] = jnp.dot(h2, w3[...], preferred_element_type=jnp.float32) + b3[...]


@functools.lru_cache(maxsize=None)
def _make_mlp(B, W, D, H1, H2, BLK):
    grid = B // BLK
    full = lambda i: (0, 0)
    return pl.pallas_call(
        _mlp_body,
        grid=(grid,),
        in_specs=[
            pl.BlockSpec((BLK, W), lambda i: (i, 0)),
            pl.BlockSpec((BLK, W), lambda i: (i, 0)),
            pl.BlockSpec((BLK, 1), lambda i: (i, 0)),
            pl.BlockSpec((BLK, 1), lambda i: (i, 0)),
            pl.BlockSpec((D, H1), full),
            pl.BlockSpec((D, H1), full),
            pl.BlockSpec((1, H1), full),
            pl.BlockSpec((H1, H2), full),
            pl.BlockSpec((1, H2), full),
            pl.BlockSpec((H2, 1), full),
            pl.BlockSpec((1, 1), full),
        ],
        out_specs=pl.BlockSpec((BLK, 1), lambda i: (i, 0)),
        out_shape=jax.ShapeDtypeStruct((B, 1), jnp.float32),
    )


def kernel(userId, productId, user_table, product_table, W1, b1, W2, b2, W3, b3):
    B = userId.shape[0]
    VU, D = user_table.shape
    VP = product_table.shape[0]
    H1 = W1.shape[1]
    H2 = W2.shape[1]
    W = _PACK * D

    upack, QU = _make_pack(D, VU)
    ppack, QP = _make_pack(D, VP)
    utab = upack(user_table.T)
    ptab = ppack(product_table.T)

    uid = userId.astype(jnp.int32)
    pid = productId.astype(jnp.int32)
    uw, pw = _make_gather(B, W)(uid % QU, pid % QP, utab, ptab)

    mlp = _make_mlp(B, W, D, H1, H2, BLK=2048)
    return mlp(uw, pw, (uid // QU)[:, None], (pid // QP)[:, None],
               W1[:D], W1[D:], b1[None, :], W2, b2[None, :], W3, b3[None, :])


# speedup vs baseline: 1.9309x; 1.0564x over previous
import functools

import jax
import jax.numpy as jnp
from jax import lax
from jax.experimental import pallas as pl
from jax.experimental.pallas import tpu as pltpu
from jax.experimental.pallas import tpu_sc as plsc

_PACK = 4
_PBLK = 4096
_CHUNK = 16


def _pack_body(s0, s1, s2, s3, dst):
    for a, s in enumerate((s0, s1, s2, s3)):
        eye = jnp.eye(s.shape[0], dtype=jnp.bfloat16)
        dst[:, a * s.shape[0]:(a + 1) * s.shape[0]] = jax.lax.dot_general(
            s[...].astype(jnp.bfloat16), eye, (((0,), (0,)), ((), ())),
            preferred_element_type=jnp.float32)


@functools.lru_cache(maxsize=None)
def _make_pack(D, V):
    grid = (V + _PACK * _PBLK - 1) // (_PACK * _PBLK)
    Q = grid * _PBLK
    last = (V + _PBLK - 1) // _PBLK - 1

    def _imap(a, i):
        return (0, jnp.minimum(i + a * grid, last))

    specs = [
        pl.BlockSpec((D, _PBLK), functools.partial(_imap, a))
        for a in range(_PACK)
    ]
    call = pl.pallas_call(
        _pack_body,
        grid=(grid,),
        in_specs=specs,
        out_specs=pl.BlockSpec((_PBLK, _PACK * D), lambda i: (i, 0)),
        out_shape=jax.ShapeDtypeStruct((Q, _PACK * D), jnp.float32),
    )

    def pack(tabT):
        return call(tabT, tabT, tabT, tabT)

    return pack, Q


@functools.lru_cache(maxsize=None)
def _make_gather(B, W):
    info = plsc.get_sparse_core_info()
    NC, NS = info.num_cores, info.num_subcores
    NW = NC * NS
    assert B % NW == 0
    b_per_w = B // NW
    assert b_per_w % _CHUNK == 0

    mesh = plsc.VectorSubcoreMesh(core_axis_name="c", subcore_axis_name="s")

    @functools.partial(
        pl.kernel,
        mesh=mesh,
        out_type=(
            jax.ShapeDtypeStruct((B, W), jnp.float32),
            jax.ShapeDtypeStruct((B, W), jnp.float32),
        ),
        scratch_types=[
            pltpu.VMEM((b_per_w,), jnp.int32),
            pltpu.VMEM((b_per_w, W), jnp.float32),
            pltpu.SemaphoreType.DMA,
        ],
    )
    def gather(uid_hbm, pid_hbm, utab_hbm, ptab_hbm, uout_hbm, pout_hbm,
               idx_s, rows_v, sem):
        wid = lax.axis_index("s") * NC + lax.axis_index("c")
        base = wid * b_per_w

        nchunk = b_per_w // _CHUNK

        def one_table(id_hbm, tab_hbm, out_hbm):
            pltpu.sync_copy(id_hbm.at[pl.ds(base, b_per_w)], idx_s)

            def fire(off):
                idx16 = idx_s[pl.ds(off, _CHUNK)]
                for j in range(_CHUNK):
                    pltpu.async_copy(
                        tab_hbm.at[pl.ds(idx16[j], 1)],
                        rows_v.at[pl.ds(off + j, 1)], sem)

            def drain(off):
                pltpu.make_async_copy(
                    tab_hbm.at[pl.ds(0, _CHUNK)],
                    rows_v.at[pl.ds(off, _CHUNK)], sem).wait()

            fire(0)

            def chunk(c, _):
                fire(c * _CHUNK)
                drain((c - 1) * _CHUNK)
                return ()

            lax.fori_loop(1, nchunk, chunk, ())
            drain((nchunk - 1) * _CHUNK)
            pltpu.sync_copy(rows_v, out_hbm.at[pl.ds(base, b_per_w)])

        one_table(uid_hbm, utab_hbm, uout_hbm)
        one_table(pid_hbm, ptab_hbm, pout_hbm)

    return gather


def _select(w, a, D):
    x = jnp.where(a == 0, w[:, 0:D], 0.0)
    for k in range(1, _PACK):
        x += jnp.where(a == k, w[:, k * D:(k + 1) * D], 0.0)
    return x


def _mlp_body(uw, pw, ua, pa, w1u, w1p, b1, w2, b2, w3, b3, out):
    D = w1u.shape[0]
    u = _select(uw[...], ua[...], D).astype(jnp.bfloat16)
    p = _select(pw[...], pa[...], D).astype(jnp.bfloat16)
    h1 = jnp.dot(u, w1u[...].astype(jnp.bfloat16),
                 preferred_element_type=jnp.float32)
    h1 += jnp.dot(p, w1p[...].astype(jnp.bfloat16),
                  preferred_element_type=jnp.float32)
    h1 = jnp.maximum(h1 + b1[...], 0.0)
    h2 = jnp.maximum(
        jnp.dot(h1, w2[...], preferred_element_type=jnp.float32) + b2[...], 0.0)
    out[...] = jnp.dot(h2, w3[...], preferred_element_type=jnp.float32) + b3[...]


@functools.lru_cache(maxsize=None)
def _make_mlp(B, W, D, H1, H2, BLK):
    grid = B // BLK
    full = lambda i: (0, 0)
    return pl.pallas_call(
        _mlp_body,
        grid=(grid,),
        in_specs=[
            pl.BlockSpec((BLK, W), lambda i: (i, 0)),
            pl.BlockSpec((BLK, W), lambda i: (i, 0)),
            pl.BlockSpec((BLK, 1), lambda i: (i, 0)),
            pl.BlockSpec((BLK, 1), lambda i: (i, 0)),
            pl.BlockSpec((D, H1), full),
            pl.BlockSpec((D, H1), full),
            pl.BlockSpec((1, H1), full),
            pl.BlockSpec((H1, H2), full),
            pl.BlockSpec((1, H2), full),
            pl.BlockSpec((H2, 1), full),
            pl.BlockSpec((1, 1), full),
        ],
        out_specs=pl.BlockSpec((BLK, 1), lambda i: (i, 0)),
        out_shape=jax.ShapeDtypeStruct((B, 1), jnp.float32),
    )


def kernel(userId, productId, user_table, product_table, W1, b1, W2, b2, W3, b3):
    B = userId.shape[0]
    VU, D = user_table.shape
    VP = product_table.shape[0]
    H1 = W1.shape[1]
    H2 = W2.shape[1]
    W = _PACK * D

    upack, QU = _make_pack(D, VU)
    ppack, QP = _make_pack(D, VP)
    utab = upack(user_table.T)
    ptab = ppack(product_table.T)

    uid = userId.astype(jnp.int32)
    pid = productId.astype(jnp.int32)
    uw, pw = _make_gather(B, W)(uid % QU, pid % QP, utab, ptab)

    mlp = _make_mlp(B, W, D, H1, H2, BLK=2048)
    return mlp(uw, pw, (uid // QU)[:, None], (pid // QP)[:, None],
               W1[:D], W1[D:], b1[None, :], W2, b2[None, :], W3, b3[None, :])


# speedup vs baseline: 2.1527x; 1.1149x over previous
import functools

import jax
import jax.numpy as jnp
from jax import lax
from jax.experimental import pallas as pl
from jax.experimental.pallas import tpu as pltpu
from jax.experimental.pallas import tpu_sc as plsc

_PACK = 4
_PBLK = 4096
_CHUNK = 32


def _pack_body(s0, s1, s2, s3, dst):
    for a, s in enumerate((s0, s1, s2, s3)):
        eye = jnp.eye(s.shape[0], dtype=jnp.bfloat16)
        dst[:, a * s.shape[0]:(a + 1) * s.shape[0]] = jax.lax.dot_general(
            s[...].astype(jnp.bfloat16), eye, (((0,), (0,)), ((), ())),
            preferred_element_type=jnp.float32)


@functools.lru_cache(maxsize=None)
def _make_pack(D, V):
    grid = (V + _PACK * _PBLK - 1) // (_PACK * _PBLK)
    Q = grid * _PBLK
    last = (V + _PBLK - 1) // _PBLK - 1

    def _imap(a, i):
        return (0, jnp.minimum(i + a * grid, last))

    specs = [
        pl.BlockSpec((D, _PBLK), functools.partial(_imap, a))
        for a in range(_PACK)
    ]
    call = pl.pallas_call(
        _pack_body,
        grid=(grid,),
        in_specs=specs,
        out_specs=pl.BlockSpec((_PBLK, _PACK * D), lambda i: (i, 0)),
        out_shape=jax.ShapeDtypeStruct((Q, _PACK * D), jnp.float32),
    )

    def pack(tabT):
        return call(tabT, tabT, tabT, tabT)

    return pack, Q


@functools.lru_cache(maxsize=None)
def _make_gather(B, W):
    info = plsc.get_sparse_core_info()
    NC, NS = info.num_cores, info.num_subcores
    NW = NC * NS
    assert B % NW == 0
    b_per_w = B // NW
    assert b_per_w % _CHUNK == 0

    mesh = plsc.VectorSubcoreMesh(core_axis_name="c", subcore_axis_name="s")

    @functools.partial(
        pl.kernel,
        mesh=mesh,
        out_type=(
            jax.ShapeDtypeStruct((B, W), jnp.float32),
            jax.ShapeDtypeStruct((B, W), jnp.float32),
        ),
        scratch_types=[
            pltpu.VMEM((b_per_w,), jnp.int32),
            pltpu.VMEM((b_per_w, W), jnp.float32),
            pltpu.SemaphoreType.DMA,
        ],
    )
    def gather(uid_hbm, pid_hbm, utab_hbm, ptab_hbm, uout_hbm, pout_hbm,
               idx_s, rows_v, sem):
        wid = lax.axis_index("s") * NC + lax.axis_index("c")
        base = wid * b_per_w

        nchunk = b_per_w // _CHUNK

        def one_table(id_hbm, tab_hbm, out_hbm):
            pltpu.sync_copy(id_hbm.at[pl.ds(base, b_per_w)], idx_s)

            def fire(off):
                for h in range(_CHUNK // 16):
                    idx16 = idx_s[pl.ds(off + h * 16, 16)]
                    for j in range(16):
                        pltpu.async_copy(
                            tab_hbm.at[pl.ds(idx16[j], 1)],
                            rows_v.at[pl.ds(off + h * 16 + j, 1)], sem)

            def drain(off):
                pltpu.make_async_copy(
                    tab_hbm.at[pl.ds(0, _CHUNK)],
                    rows_v.at[pl.ds(off, _CHUNK)], sem).wait()

            fire(0)

            def chunk(c, _):
                fire(c * _CHUNK)
                drain((c - 1) * _CHUNK)
                return ()

            lax.fori_loop(1, nchunk, chunk, ())
            drain((nchunk - 1) * _CHUNK)
            pltpu.sync_copy(rows_v, out_hbm.at[pl.ds(base, b_per_w)])

        one_table(uid_hbm, utab_hbm, uout_hbm)
        one_table(pid_hbm, ptab_hbm, pout_hbm)

    return gather


def _mlp_body(uw, pw, ua, pa, w1u, w1p, b1, w2, b2, w3, b3, out):
    W = uw.shape[1]
    D = W // _PACK
    lane_q = jax.lax.broadcasted_iota(jnp.int32, (1, W), 1) // D
    um = jnp.where(ua[...] == lane_q, uw[...], 0.0).astype(jnp.bfloat16)
    pm = jnp.where(pa[...] == lane_q, pw[...], 0.0).astype(jnp.bfloat16)
    h1 = jnp.dot(um, w1u[...], preferred_element_type=jnp.float32)
    h1 += jnp.dot(pm, w1p[...], preferred_element_type=jnp.float32)
    h1 = jnp.maximum(h1 + b1[...], 0.0)
    h2 = jnp.maximum(
        jnp.dot(h1, w2[...], preferred_element_type=jnp.float32) + b2[...], 0.0)
    out[...] = jnp.dot(h2, w3[...], preferred_element_type=jnp.float32) + b3[...]


@functools.lru_cache(maxsize=None)
def _make_mlp(B, W, D, H1, H2, BLK):
    grid = B // BLK
    full = lambda i: (0, 0)
    return pl.pallas_call(
        _mlp_body,
        grid=(grid,),
        in_specs=[
            pl.BlockSpec((BLK, W), lambda i: (i, 0)),
            pl.BlockSpec((BLK, W), lambda i: (i, 0)),
            pl.BlockSpec((BLK, 1), lambda i: (i, 0)),
            pl.BlockSpec((BLK, 1), lambda i: (i, 0)),
            pl.BlockSpec((W, H1), full),
            pl.BlockSpec((W, H1), full),
            pl.BlockSpec((1, H1), full),
            pl.BlockSpec((H1, H2), full),
            pl.BlockSpec((1, H2), full),
            pl.BlockSpec((H2, 1), full),
            pl.BlockSpec((1, 1), full),
        ],
        out_specs=pl.BlockSpec((BLK, 1), lambda i: (i, 0)),
        out_shape=jax.ShapeDtypeStruct((B, 1), jnp.float32),
    )


def kernel(userId, productId, user_table, product_table, W1, b1, W2, b2, W3, b3):
    B = userId.shape[0]
    VU, D = user_table.shape
    VP = product_table.shape[0]
    H1 = W1.shape[1]
    H2 = W2.shape[1]
    W = _PACK * D

    upack, QU = _make_pack(D, VU)
    ppack, QP = _make_pack(D, VP)
    utab = upack(user_table.T)
    ptab = ppack(product_table.T)

    uid = userId.astype(jnp.int32)
    pid = productId.astype(jnp.int32)
    uw, pw = _make_gather(B, W)(uid % QU, pid % QP, utab, ptab)

    w1ux = jnp.tile(W1[:D], (_PACK, 1)).astype(jnp.bfloat16)
    w1px = jnp.tile(W1[D:], (_PACK, 1)).astype(jnp.bfloat16)
    mlp = _make_mlp(B, W, D, H1, H2, BLK=2048)
    return mlp(uw, pw, (uid // QU)[:, None], (pid // QP)[:, None],
               w1ux, w1px, b1[None, :], W2, b2[None, :], W3, b3[None, :])


# speedup vs baseline: 2.2180x; 1.0303x over previous
import functools

import jax
import jax.numpy as jnp
from jax import lax
from jax.experimental import pallas as pl
from jax.experimental.pallas import tpu as pltpu
from jax.experimental.pallas import tpu_sc as plsc

_PACK = 4
_PBLK = 4096
_CHUNK = 32


def _pack_body(s0, s1, s2, s3, dst):
    for a, s in enumerate((s0, s1, s2, s3)):
        eye = jnp.eye(s.shape[0], dtype=jnp.bfloat16)
        dst[:, a * s.shape[0]:(a + 1) * s.shape[0]] = jax.lax.dot_general(
            s[...].astype(jnp.bfloat16), eye, (((0,), (0,)), ((), ())),
            preferred_element_type=jnp.float32)


@functools.lru_cache(maxsize=None)
def _make_pack(D, V):
    grid = (V + _PACK * _PBLK - 1) // (_PACK * _PBLK)
    Q = grid * _PBLK
    last = (V + _PBLK - 1) // _PBLK - 1

    def _imap(a, i):
        return (0, jnp.minimum(i + a * grid, last))

    specs = [
        pl.BlockSpec((D, _PBLK), functools.partial(_imap, a))
        for a in range(_PACK)
    ]
    call = pl.pallas_call(
        _pack_body,
        grid=(grid,),
        in_specs=specs,
        out_specs=pl.BlockSpec((_PBLK, _PACK * D), lambda i: (i, 0)),
        out_shape=jax.ShapeDtypeStruct((Q, _PACK * D), jnp.float32),
    )

    def pack(tabT):
        return call(tabT, tabT, tabT, tabT)

    return pack, Q


@functools.lru_cache(maxsize=None)
def _make_gather(B, W):
    info = plsc.get_sparse_core_info()
    NC, NS = info.num_cores, info.num_subcores
    NW = NC * NS
    assert B % NW == 0
    b_per_w = B // NW
    assert b_per_w % _CHUNK == 0

    mesh = plsc.VectorSubcoreMesh(core_axis_name="c", subcore_axis_name="s")

    @functools.partial(
        pl.kernel,
        mesh=mesh,
        out_type=jax.ShapeDtypeStruct((B, W), jnp.float32),
        scratch_types=[
            pltpu.VMEM((b_per_w,), jnp.int32),
            pltpu.VMEM((b_per_w, W), jnp.float32),
            pltpu.SemaphoreType.DMA,
        ],
    )
    def gather(id_hbm, tab_hbm, out_hbm, idx_s, rows_v, sem):
        wid = lax.axis_index("s") * NC + lax.axis_index("c")
        base = wid * b_per_w
        nchunk = b_per_w // _CHUNK

        pltpu.sync_copy(id_hbm.at[pl.ds(base, b_per_w)], idx_s)

        def fire(off):
            for h in range(_CHUNK // 16):
                idx16 = idx_s[pl.ds(off + h * 16, 16)]
                for j in range(16):
                    pltpu.async_copy(
                        tab_hbm.at[pl.ds(idx16[j], 1)],
                        rows_v.at[pl.ds(off + h * 16 + j, 1)], sem)

        def drain(off):
            pltpu.make_async_copy(
                tab_hbm.at[pl.ds(0, _CHUNK)],
                rows_v.at[pl.ds(off, _CHUNK)], sem).wait()

        fire(0)

        def chunk(c, _):
            fire(c * _CHUNK)
            drain((c - 1) * _CHUNK)
            return ()

        lax.fori_loop(1, nchunk, chunk, ())
        drain((nchunk - 1) * _CHUNK)
        pltpu.sync_copy(rows_v, out_hbm.at[pl.ds(base, b_per_w)])

    return gather


def _mlp_body(uw, pw, ua, pa, w1u, w1p, b1, w2, b2, w3, b3, out):
    W = uw.shape[1]
    D = W // _PACK
    lane_q = jax.lax.broadcasted_iota(jnp.int32, (1, W), 1) // D
    um = jnp.where(ua[...] == lane_q, uw[...], 0.0).astype(jnp.bfloat16)
    pm = jnp.where(pa[...] == lane_q, pw[...], 0.0).astype(jnp.bfloat16)
    h1 = jnp.dot(um, w1u[...], preferred_element_type=jnp.float32)
    h1 += jnp.dot(pm, w1p[...], preferred_element_type=jnp.float32)
    h1 = jnp.maximum(h1 + b1[...], 0.0)
    h2 = jnp.maximum(
        jnp.dot(h1, w2[...], preferred_element_type=jnp.float32) + b2[...], 0.0)
    out[...] = jnp.dot(h2, w3[...], preferred_element_type=jnp.float32) + b3[...]


@functools.lru_cache(maxsize=None)
def _make_mlp(B, W, D, H1, H2, BLK):
    grid = B // BLK
    full = lambda i: (0, 0)
    return pl.pallas_call(
        _mlp_body,
        grid=(grid,),
        in_specs=[
            pl.BlockSpec((BLK, W), lambda i: (i, 0)),
            pl.BlockSpec((BLK, W), lambda i: (i, 0)),
            pl.BlockSpec((BLK, 1), lambda i: (i, 0)),
            pl.BlockSpec((BLK, 1), lambda i: (i, 0)),
            pl.BlockSpec((W, H1), full),
            pl.BlockSpec((W, H1), full),
            pl.BlockSpec((1, H1), full),
            pl.BlockSpec((H1, H2), full),
            pl.BlockSpec((1, H2), full),
            pl.BlockSpec((H2, 1), full),
            pl.BlockSpec((1, 1), full),
        ],
        out_specs=pl.BlockSpec((BLK, 1), lambda i: (i, 0)),
        out_shape=jax.ShapeDtypeStruct((B, 1), jnp.float32),
    )


def kernel(userId, productId, user_table, product_table, W1, b1, W2, b2, W3, b3):
    B = userId.shape[0]
    VU, D = user_table.shape
    VP = product_table.shape[0]
    H1 = W1.shape[1]
    H2 = W2.shape[1]
    W = _PACK * D

    upack, QU = _make_pack(D, VU)
    ppack, QP = _make_pack(D, VP)
    utab = upack(user_table.T)
    ptab = ppack(product_table.T)

    uid = userId.astype(jnp.int32)
    pid = productId.astype(jnp.int32)
    gather = _make_gather(B, W)
    uw = gather(uid % QU, utab)
    pw = gather(pid % QP, ptab)

    w1ux = jnp.tile(W1[:D], (_PACK, 1)).astype(jnp.bfloat16)
    w1px = jnp.tile(W1[D:], (_PACK, 1)).astype(jnp.bfloat16)
    mlp = _make_mlp(B, W, D, H1, H2, BLK=2048)
    return mlp(uw, pw, (uid // QU)[:, None], (pid // QP)[:, None],
               w1ux, w1px, b1[None, :], W2, b2[None, :], W3, b3[None, :])


# speedup vs baseline: 2.2446x; 1.0120x over previous
import functools

import jax
import jax.numpy as jnp
from jax import lax
from jax.experimental import pallas as pl
from jax.experimental.pallas import tpu as pltpu
from jax.experimental.pallas import tpu_sc as plsc

_PACK = 4
_PBLK = 8192
_CHUNK = 32


def _pack_body(s0, s1, s2, s3, dst):
    for a, s in enumerate((s0, s1, s2, s3)):
        eye = jnp.eye(s.shape[0], dtype=jnp.bfloat16)
        dst[:, a * s.shape[0]:(a + 1) * s.shape[0]] = jax.lax.dot_general(
            s[...].astype(jnp.bfloat16), eye, (((0,), (0,)), ((), ())),
            preferred_element_type=jnp.float32)


@functools.lru_cache(maxsize=None)
def _make_pack(D, V):
    grid = (V + _PACK * _PBLK - 1) // (_PACK * _PBLK)
    Q = grid * _PBLK
    last = (V + _PBLK - 1) // _PBLK - 1

    def _imap(a, i):
        return (0, jnp.minimum(i + a * grid, last))

    specs = [
        pl.BlockSpec((D, _PBLK), functools.partial(_imap, a))
        for a in range(_PACK)
    ]
    call = pl.pallas_call(
        _pack_body,
        grid=(grid,),
        in_specs=specs,
        out_specs=pl.BlockSpec((_PBLK, _PACK * D), lambda i: (i, 0)),
        out_shape=jax.ShapeDtypeStruct((Q, _PACK * D), jnp.float32),
    )

    def pack(tabT):
        return call(tabT, tabT, tabT, tabT)

    return pack, Q


@functools.lru_cache(maxsize=None)
def _make_gather(B, W):
    info = plsc.get_sparse_core_info()
    NC, NS = info.num_cores, info.num_subcores
    NW = NC * NS
    assert B % NW == 0
    b_per_w = B // NW
    assert b_per_w % _CHUNK == 0

    mesh = plsc.VectorSubcoreMesh(core_axis_name="c", subcore_axis_name="s")

    @functools.partial(
        pl.kernel,
        mesh=mesh,
        out_type=jax.ShapeDtypeStruct((B, W), jnp.float32),
        scratch_types=[
            pltpu.VMEM((b_per_w,), jnp.int32),
            pltpu.VMEM((b_per_w, W), jnp.float32),
            pltpu.SemaphoreType.DMA,
        ],
    )
    def gather(id_hbm, tab_hbm, out_hbm, idx_s, rows_v, sem):
        wid = lax.axis_index("s") * NC + lax.axis_index("c")
        base = wid * b_per_w
        nchunk = b_per_w // _CHUNK

        pltpu.sync_copy(id_hbm.at[pl.ds(base, b_per_w)], idx_s)

        def fire(off):
            for h in range(_CHUNK // 16):
                idx16 = idx_s[pl.ds(off + h * 16, 16)]
                for j in range(16):
                    pltpu.async_copy(
                        tab_hbm.at[pl.ds(idx16[j], 1)],
                        rows_v.at[pl.ds(off + h * 16 + j, 1)], sem)

        def drain(off):
            pltpu.make_async_copy(
                tab_hbm.at[pl.ds(0, _CHUNK)],
                rows_v.at[pl.ds(off, _CHUNK)], sem).wait()

        fire(0)

        def chunk(c, _):
            fire(c * _CHUNK)
            drain((c - 1) * _CHUNK)
            return ()

        lax.fori_loop(1, nchunk, chunk, ())
        drain((nchunk - 1) * _CHUNK)
        pltpu.sync_copy(rows_v, out_hbm.at[pl.ds(base, b_per_w)])

    return gather


def _mlp_body(uw, pw, ua, pa, w1u, w1p, b1, w2, b2, w3, b3, out):
    W = uw.shape[1]
    D = W // _PACK
    lane_q = jax.lax.broadcasted_iota(jnp.int32, (1, W), 1) // D
    um = jnp.where(ua[...] == lane_q, uw[...], 0.0).astype(jnp.bfloat16)
    pm = jnp.where(pa[...] == lane_q, pw[...], 0.0).astype(jnp.bfloat16)
    h1 = jnp.dot(um, w1u[...], preferred_element_type=jnp.float32)
    h1 += jnp.dot(pm, w1p[...], preferred_element_type=jnp.float32)
    h1 = jnp.maximum(h1 + b1[...], 0.0)
    h2 = jnp.maximum(
        jnp.dot(h1, w2[...], preferred_element_type=jnp.float32) + b2[...], 0.0)
    out[...] = jnp.dot(h2, w3[...], preferred_element_type=jnp.float32) + b3[...]


@functools.lru_cache(maxsize=None)
def _make_mlp(B, W, D, H1, H2, BLK):
    grid = B // BLK
    full = lambda i: (0, 0)
    return pl.pallas_call(
        _mlp_body,
        grid=(grid,),
        in_specs=[
            pl.BlockSpec((BLK, W), lambda i: (i, 0)),
            pl.BlockSpec((BLK, W), lambda i: (i, 0)),
            pl.BlockSpec((BLK, 1), lambda i: (i, 0)),
            pl.BlockSpec((BLK, 1), lambda i: (i, 0)),
            pl.BlockSpec((W, H1), full),
            pl.BlockSpec((W, H1), full),
            pl.BlockSpec((1, H1), full),
            pl.BlockSpec((H1, H2), full),
            pl.BlockSpec((1, H2), full),
            pl.BlockSpec((H2, 1), full),
            pl.BlockSpec((1, 1), full),
        ],
        out_specs=pl.BlockSpec((BLK, 1), lambda i: (i, 0)),
        out_shape=jax.ShapeDtypeStruct((B, 1), jnp.float32),
    )


def kernel(userId, productId, user_table, product_table, W1, b1, W2, b2, W3, b3):
    B = userId.shape[0]
    VU, D = user_table.shape
    VP = product_table.shape[0]
    H1 = W1.shape[1]
    H2 = W2.shape[1]
    W = _PACK * D

    upack, QU = _make_pack(D, VU)
    ppack, QP = _make_pack(D, VP)
    utab = upack(user_table.T)
    ptab = ppack(product_table.T)

    uid = userId.astype(jnp.int32)
    pid = productId.astype(jnp.int32)
    gather = _make_gather(B, W)
    uw = gather(uid % QU, utab)
    pw = gather(pid % QP, ptab)

    w1ux = jnp.tile(W1[:D], (_PACK, 1)).astype(jnp.bfloat16)
    w1px = jnp.tile(W1[D:], (_PACK, 1)).astype(jnp.bfloat16)
    mlp = _make_mlp(B, W, D, H1, H2, BLK=4096)
    return mlp(uw, pw, (uid // QU)[:, None], (pid // QP)[:, None],
               w1ux, w1px, b1[None, :], W2, b2[None, :], W3, b3[None, :])


# speedup vs baseline: 2.7466x; 1.2236x over previous
import functools

import jax
import jax.numpy as jnp
from jax import lax
from jax.experimental import pallas as pl
from jax.experimental.pallas import tpu as pltpu
from jax.experimental.pallas import tpu_sc as plsc

_PACK = 8
_PBLK = 4096
_CHUNK = 32


def _pack_body(*refs):
    srcs, dst = refs[:-1], refs[-1]
    D = srcs[0].shape[0]
    eye = jnp.eye(D, dtype=jnp.bfloat16)
    cols = []
    for s in srcs:
        cols.append(jax.lax.dot_general(
            s[...].astype(jnp.bfloat16), eye, (((0,), (0,)), ((), ())),
            preferred_element_type=jnp.float32).astype(jnp.bfloat16))
    t = jnp.concatenate(cols, axis=1)
    half = t.shape[1] // 2
    lo = jax.lax.bitcast_convert_type(t[:, :half], jnp.uint16).astype(jnp.uint32)
    hi = jax.lax.bitcast_convert_type(t[:, half:], jnp.uint16).astype(jnp.uint32)
    dst[...] = lo | (hi << 16)


@functools.lru_cache(maxsize=None)
def _make_pack(D, V):
    grid = (V + _PACK * _PBLK - 1) // (_PACK * _PBLK)
    Q = grid * _PBLK
    last = (V + _PBLK - 1) // _PBLK - 1

    def _imap(a, i):
        return (0, jnp.minimum(i + a * grid, last))

    specs = [
        pl.BlockSpec((D, _PBLK), functools.partial(_imap, a))
        for a in range(_PACK)
    ]
    call = pl.pallas_call(
        _pack_body,
        grid=(grid,),
        in_specs=specs,
        out_specs=pl.BlockSpec((_PBLK, _PACK * D // 2), lambda i: (i, 0)),
        out_shape=jax.ShapeDtypeStruct((Q, _PACK * D // 2), jnp.uint32),
    )

    def pack(tabT):
        return call(*([tabT] * _PACK))

    return pack, Q


@functools.lru_cache(maxsize=None)
def _make_gather(B, W):
    info = plsc.get_sparse_core_info()
    NC, NS = info.num_cores, info.num_subcores
    NW = NC * NS
    assert B % NW == 0
    b_per_w = B // NW
    assert b_per_w % _CHUNK == 0

    mesh = plsc.VectorSubcoreMesh(core_axis_name="c", subcore_axis_name="s")

    @functools.partial(
        pl.kernel,
        mesh=mesh,
        out_type=jax.ShapeDtypeStruct((B, W), jnp.uint32),
        scratch_types=[
            pltpu.VMEM((b_per_w,), jnp.int32),
            pltpu.VMEM((b_per_w, W), jnp.uint32),
            pltpu.SemaphoreType.DMA,
        ],
    )
    def gather(id_hbm, tab_hbm, out_hbm, idx_s, rows_v, sem):
        wid = lax.axis_index("s") * NC + lax.axis_index("c")
        base = wid * b_per_w
        nchunk = b_per_w // _CHUNK

        pltpu.sync_copy(id_hbm.at[pl.ds(base, b_per_w)], idx_s)

        def fire(off):
            for h in range(_CHUNK // 16):
                idx16 = idx_s[pl.ds(off + h * 16, 16)]
                for j in range(16):
                    pltpu.async_copy(
                        tab_hbm.at[pl.ds(idx16[j], 1)],
                        rows_v.at[pl.ds(off + h * 16 + j, 1)], sem)

        def drain(off):
            pltpu.make_async_copy(
                tab_hbm.at[pl.ds(0, _CHUNK)],
                rows_v.at[pl.ds(off, _CHUNK)], sem).wait()

        fire(0)

        def chunk(c, _):
            fire(c * _CHUNK)
            drain((c - 1) * _CHUNK)
            return ()

        lax.fori_loop(1, nchunk, chunk, ())
        drain((nchunk - 1) * _CHUNK)
        pltpu.sync_copy(rows_v, out_hbm.at[pl.ds(base, b_per_w)])

    return gather


def _mlp_body(uw, pw, ua, pa, w1u, w1p, b1, w2, b2, w3, b3, out):
    W = uw.shape[1]
    D = 2 * W // _PACK
    lq = jax.lax.broadcasted_iota(jnp.int32, (1, W), 1) // D

    def to_bf16(w, a):
        lo = jax.lax.bitcast_convert_type(
            (w & jnp.uint32(0xFFFF)).astype(jnp.uint16), jnp.bfloat16)
        hi = jax.lax.bitcast_convert_type(
            (w >> 16).astype(jnp.uint16), jnp.bfloat16)
        zero = jnp.zeros_like(lo)
        return jnp.concatenate([
            jnp.where(a == lq, lo, zero),
            jnp.where(a == _PACK // 2 + lq, hi, zero),
        ], axis=1)

    um = to_bf16(uw[...], ua[...])
    pm = to_bf16(pw[...], pa[...])
    h1 = jnp.dot(um, w1u[...], preferred_element_type=jnp.float32)
    h1 += jnp.dot(pm, w1p[...], preferred_element_type=jnp.float32)
    h1 = jnp.maximum(h1 + b1[...], 0.0)
    h2 = jnp.maximum(
        jnp.dot(h1, w2[...], preferred_element_type=jnp.float32) + b2[...], 0.0)
    out[...] = jnp.dot(h2, w3[...], preferred_element_type=jnp.float32) + b3[...]


@functools.lru_cache(maxsize=None)
def _make_mlp(B, W, D, H1, H2, BLK):
    grid = B // BLK
    full = lambda i: (0, 0)
    return pl.pallas_call(
        _mlp_body,
        grid=(grid,),
        in_specs=[
            pl.BlockSpec((BLK, W), lambda i: (i, 0)),
            pl.BlockSpec((BLK, W), lambda i: (i, 0)),
            pl.BlockSpec((BLK, 1), lambda i: (i, 0)),
            pl.BlockSpec((BLK, 1), lambda i: (i, 0)),
            pl.BlockSpec((2 * W, H1), full),
            pl.BlockSpec((2 * W, H1), full),
            pl.BlockSpec((1, H1), full),
            pl.BlockSpec((H1, H2), full),
            pl.BlockSpec((1, H2), full),
            pl.BlockSpec((H2, 1), full),
            pl.BlockSpec((1, 1), full),
        ],
        out_specs=pl.BlockSpec((BLK, 1), lambda i: (i, 0)),
        out_shape=jax.ShapeDtypeStruct((B, 1), jnp.float32),
    )


def kernel(userId, productId, user_table, product_table, W1, b1, W2, b2, W3, b3):
    B = userId.shape[0]
    VU, D = user_table.shape
    VP = product_table.shape[0]
    H1 = W1.shape[1]
    H2 = W2.shape[1]
    W = _PACK * D // 2

    upack, QU = _make_pack(D, VU)
    ppack, QP = _make_pack(D, VP)
    utab = upack(user_table.T)
    ptab = ppack(product_table.T)

    uid = userId.astype(jnp.int32)
    pid = productId.astype(jnp.int32)
    gather = _make_gather(B, W)
    uw = gather(uid % QU, utab)
    pw = gather(pid % QP, ptab)

    w1ux = jnp.tile(W1[:D], (_PACK, 1)).astype(jnp.bfloat16)
    w1px = jnp.tile(W1[D:], (_PACK, 1)).astype(jnp.bfloat16)
    mlp = _make_mlp(B, W, D, H1, H2, BLK=4096)
    return mlp(uw, pw, (uid // QU)[:, None], (pid // QP)[:, None],
               w1ux, w1px, b1[None, :], W2, b2[None, :], W3, b3[None, :])


# speedup vs baseline: 2.8477x; 1.0368x over previous
import functools

import jax
import jax.numpy as jnp
from jax import lax
from jax.experimental import pallas as pl
from jax.experimental.pallas import tpu as pltpu
from jax.experimental.pallas import tpu_sc as plsc

_PACK = 8
_PBLK = 4096
_CHUNK = 32


def _pack_body(*refs):
    srcs, dst = refs[:-1], refs[-1]
    D = srcs[0].shape[0]
    eye = jnp.eye(D, dtype=jnp.bfloat16)
    cols = []
    for s in srcs:
        cols.append(jax.lax.dot_general(
            s[...].astype(jnp.bfloat16), eye, (((0,), (0,)), ((), ())),
            preferred_element_type=jnp.float32).astype(jnp.bfloat16))
    t = jnp.concatenate(cols, axis=1)
    half = t.shape[1] // 2
    lo = jax.lax.bitcast_convert_type(t[:, :half], jnp.uint16).astype(jnp.uint32)
    hi = jax.lax.bitcast_convert_type(t[:, half:], jnp.uint16).astype(jnp.uint32)
    dst[...] = lo | (hi << 16)


@functools.lru_cache(maxsize=None)
def _make_pack(D, V):
    grid = (V + _PACK * _PBLK - 1) // (_PACK * _PBLK)
    Q = grid * _PBLK
    last = (V + _PBLK - 1) // _PBLK - 1

    def _imap(a, i):
        return (0, jnp.minimum(i + a * grid, last))

    specs = [
        pl.BlockSpec((D, _PBLK), functools.partial(_imap, a))
        for a in range(_PACK)
    ]
    call = pl.pallas_call(
        _pack_body,
        grid=(grid,),
        in_specs=specs,
        out_specs=pl.BlockSpec((_PBLK, _PACK * D // 2), lambda i: (i, 0)),
        out_shape=jax.ShapeDtypeStruct((Q, _PACK * D // 2), jnp.uint32),
    )

    def pack(tabT):
        return call(*([tabT] * _PACK))

    return pack, Q


@functools.lru_cache(maxsize=None)
def _make_gather(B, W):
    info = plsc.get_sparse_core_info()
    NC, NS = info.num_cores, info.num_subcores
    NW = NC * NS
    assert B % NW == 0
    b_per_w = B // NW
    assert b_per_w % _CHUNK == 0

    mesh = plsc.VectorSubcoreMesh(core_axis_name="c", subcore_axis_name="s")

    @functools.partial(
        pl.kernel,
        mesh=mesh,
        out_type=jax.ShapeDtypeStruct((B, W), jnp.uint32),
        scratch_types=[
            pltpu.VMEM((b_per_w,), jnp.int32),
            pltpu.VMEM((b_per_w, W), jnp.uint32),
            pltpu.SemaphoreType.DMA,
        ],
    )
    def gather(id_hbm, tab_hbm, out_hbm, idx_s, rows_v, sem):
        wid = lax.axis_index("s") * NC + lax.axis_index("c")
        base = wid * b_per_w
        nchunk = b_per_w // _CHUNK

        pltpu.sync_copy(id_hbm.at[pl.ds(base, b_per_w)], idx_s)

        def fire(off):
            for h in range(_CHUNK // 16):
                idx16 = idx_s[pl.ds(off + h * 16, 16)]
                for j in range(16):
                    pltpu.async_copy(
                        tab_hbm.at[pl.ds(idx16[j], 1)],
                        rows_v.at[pl.ds(off + h * 16 + j, 1)], sem)

        def drain(off):
            pltpu.make_async_copy(
                tab_hbm.at[pl.ds(0, _CHUNK)],
                rows_v.at[pl.ds(off, _CHUNK)], sem).wait()

        fire(0)

        def chunk(c, _):
            fire(c * _CHUNK)
            drain((c - 1) * _CHUNK)
            return ()

        lax.fori_loop(1, nchunk, chunk, ())
        drain((nchunk - 1) * _CHUNK)
        pltpu.sync_copy(rows_v, out_hbm.at[pl.ds(base, b_per_w)])

    return gather


def _mlp_body(uw, pw, ua, pa, w1u, w1p, b1, w2, b2, w3, b3, out):
    W = uw.shape[1]
    D = 2 * W // _PACK
    lq = jax.lax.broadcasted_iota(jnp.int32, (1, W), 1) // D

    def to_bf16(w, a):
        lo = jax.lax.bitcast_convert_type(
            (w & jnp.uint32(0xFFFF)).astype(jnp.uint16), jnp.bfloat16)
        hi = jax.lax.bitcast_convert_type(
            (w >> 16).astype(jnp.uint16), jnp.bfloat16)
        zero = jnp.zeros_like(lo)
        return jnp.concatenate([
            jnp.where(a == lq, lo, zero),
            jnp.where(a == _PACK // 2 + lq, hi, zero),
        ], axis=1)

    um = to_bf16(uw[...], ua[...])
    pm = to_bf16(pw[...], pa[...])
    h1 = jnp.dot(um, w1u[...], preferred_element_type=jnp.float32)
    h1 += jnp.dot(pm, w1p[...], preferred_element_type=jnp.float32)
    h1 = jnp.maximum(h1 + b1[...], 0.0)
    h2 = jnp.maximum(
        jnp.dot(h1, w2[...], preferred_element_type=jnp.float32) + b2[...], 0.0)
    out[...] = jax.lax.dot_general(
        w3[...], h2, (((0,), (1,)), ((), ())),
        preferred_element_type=jnp.float32) + b3[...]


@functools.lru_cache(maxsize=None)
def _make_mlp(B, W, D, H1, H2, BLK):
    grid = B // BLK
    full = lambda i: (0, 0)
    return pl.pallas_call(
        _mlp_body,
        grid=(grid,),
        in_specs=[
            pl.BlockSpec((BLK, W), lambda i: (i, 0)),
            pl.BlockSpec((BLK, W), lambda i: (i, 0)),
            pl.BlockSpec((BLK, 1), lambda i: (i, 0)),
            pl.BlockSpec((BLK, 1), lambda i: (i, 0)),
            pl.BlockSpec((2 * W, H1), full),
            pl.BlockSpec((2 * W, H1), full),
            pl.BlockSpec((1, H1), full),
            pl.BlockSpec((H1, H2), full),
            pl.BlockSpec((1, H2), full),
            pl.BlockSpec((H2, 1), full),
            pl.BlockSpec((1, 1), full),
        ],
        out_specs=pl.BlockSpec((1, BLK), lambda i: (0, i)),
        out_shape=jax.ShapeDtypeStruct((1, B), jnp.float32),
    )


def kernel(userId, productId, user_table, product_table, W1, b1, W2, b2, W3, b3):
    B = userId.shape[0]
    VU, D = user_table.shape
    VP = product_table.shape[0]
    H1 = W1.shape[1]
    H2 = W2.shape[1]
    W = _PACK * D // 2

    upack, QU = _make_pack(D, VU)
    ppack, QP = _make_pack(D, VP)
    uid = userId.astype(jnp.int32)
    pid = productId.astype(jnp.int32)
    gather = _make_gather(B, W)
    ptab = ppack(product_table.T)
    pw = gather(pid % QP, ptab)
    utab = upack(user_table.T)
    uw = gather(uid % QU, utab)

    w1ux = jnp.tile(W1[:D], (_PACK, 1)).astype(jnp.bfloat16)
    w1px = jnp.tile(W1[D:], (_PACK, 1)).astype(jnp.bfloat16)
    mlp = _make_mlp(B, W, D, H1, H2, BLK=4096)
    return mlp(uw, pw, (uid // QU)[:, None], (pid // QP)[:, None],
               w1ux, w1px, b1[None, :], W2, b2[None, :], W3, b3[None, :]).T
